# baseline (speedup 1.0000x reference)
import functools

import jax
import jax.numpy as jnp
from jax import lax
from jax.experimental import pallas as pl
from jax.experimental.pallas import tpu as pltpu

ALPHA = 0.1
NEG = -9e15


def _row_softmax_unnorm(e, mask):
    e = jnp.maximum(e, ALPHA * e)
    e = jnp.where(mask, e, NEG)
    m = jnp.max(e, axis=1, keepdims=True)
    ex = jnp.exp(e - m)
    s = jnp.sum(ex, axis=1, keepdims=True)
    return ex * (1.0 / s)


def _k1_body(feat_ref, w1_ref, b1_ref, adj_ref, x1_ref, adj16_ref, p1_scr):
    @pl.when(pl.program_id(0) == 0)
    def _():
        p1_scr[...] = jnp.dot(feat_ref[...], w1_ref[...],
                              preferred_element_type=jnp.float32)

    a = adj_ref[...]
    adj16_ref[...] = a.astype(jnp.bfloat16)
    x = jnp.dot(a, p1_scr[...], preferred_element_type=jnp.float32)
    x1_ref[...] = jnp.maximum(x + b1_ref[...], 0.0)


def _k1(feature, w1, b1, adj, br):
    n, feat = feature.shape
    hid = w1.shape[1]
    return pl.pallas_call(
        _k1_body,
        grid=(n // br,),
        in_specs=[
            pl.BlockSpec((n, feat), lambda i: (0, 0)),
            pl.BlockSpec((feat, hid), lambda i: (0, 0)),
            pl.BlockSpec((1, hid), lambda i: (0, 0)),
            pl.BlockSpec((br, n), lambda i: (i, 0)),
        ],
        out_specs=[
            pl.BlockSpec((br, hid), lambda i: (i, 0)),
            pl.BlockSpec((br, n), lambda i: (i, 0)),
        ],
        out_shape=[
            jax.ShapeDtypeStruct((n, hid), jnp.float32),
            jax.ShapeDtypeStruct((n, n), jnp.bfloat16),
        ],
        scratch_shapes=[pltpu.VMEM((n, hid), jnp.float32)],
    )(feature, w1, b1, adj)


def _k2_body(x1_ref, w2_ref, b2_ref, gatw_ref, a1_ref, a2_ref, adj16_ref,
             wh_ref, f1_ref, f2t_ref, p2_scr, *, heads):
    @pl.when(pl.program_id(0) == 0)
    def _():
        p2_scr[...] = jnp.dot(x1_ref[...], w2_ref[...],
                              preferred_element_type=jnp.float32)

    a = adj16_ref[...].astype(jnp.float32)
    x2 = jnp.dot(a, p2_scr[...], preferred_element_type=jnp.float32)
    x2 = x2 + b2_ref[...]
    for h in range(heads):
        wh = jnp.dot(x2, gatw_ref[h], preferred_element_type=jnp.float32)
        wh_ref[h] = wh.astype(jnp.bfloat16)
        f1_ref[h] = lax.dot_general(wh, a1_ref[h], (((1,), (1,)), ((), ())),
                                    preferred_element_type=jnp.float32)
        f2t_ref[h] = lax.dot_general(a2_ref[h], wh, (((1,), (1,)), ((), ())),
                                     preferred_element_type=jnp.float32)


def _k2(x1, w2, b2, gat_w, a1, a2, adj16, br):
    n, hid = x1.shape
    heads, feat, ghid = gat_w.shape
    return pl.pallas_call(
        functools.partial(_k2_body, heads=heads),
        grid=(n // br,),
        in_specs=[
            pl.BlockSpec((n, hid), lambda i: (0, 0)),
            pl.BlockSpec((hid, feat), lambda i: (0, 0)),
            pl.BlockSpec((1, feat), lambda i: (0, 0)),
            pl.BlockSpec((heads, feat, ghid), lambda i: (0, 0, 0)),
            pl.BlockSpec((heads, 1, ghid), lambda i: (0, 0, 0)),
            pl.BlockSpec((heads, 1, ghid), lambda i: (0, 0, 0)),
            pl.BlockSpec((br, n), lambda i: (i, 0)),
        ],
        out_specs=[
            pl.BlockSpec((heads, br, ghid), lambda i: (0, i, 0)),
            pl.BlockSpec((heads, br, 1), lambda i: (0, i, 0)),
            pl.BlockSpec((heads, 1, br), lambda i: (0, 0, i)),
        ],
        out_shape=[
            jax.ShapeDtypeStruct((heads, n, ghid), jnp.bfloat16),
            jax.ShapeDtypeStruct((heads, n, 1), jnp.float32),
            jax.ShapeDtypeStruct((heads, 1, n), jnp.float32),
        ],
        scratch_shapes=[pltpu.VMEM((n, feat), jnp.float32)],
    )(x1, w2, b2, gat_w, a1, a2, adj16)


def _k3_body(adj16_ref, wh_ref, f1_ref, f2t_ref, outw_ref, oa1_ref, oa2_ref,
             who_ref, f1o_ref, f2to_ref, *, heads):
    mask = adj16_ref[...] > 0
    cats = []
    for h in range(heads):
        att = _row_softmax_unnorm(f1_ref[h] + f2t_ref[h], mask)
        hp = jnp.dot(att, wh_ref[h], preferred_element_type=jnp.float32)
        cats.append(jnp.where(hp > 0, hp, jnp.exp(hp) - 1.0))
    xcat = jnp.concatenate(cats, axis=1)
    who = jnp.dot(xcat, outw_ref[...], preferred_element_type=jnp.float32)
    who_ref[...] = who
    f1o_ref[...] = lax.dot_general(who, oa1_ref[...], (((1,), (1,)), ((), ())),
                                   preferred_element_type=jnp.float32)
    f2to_ref[...] = lax.dot_general(oa2_ref[...], who, (((1,), (1,)), ((), ())),
                                    preferred_element_type=jnp.float32)


def _k3(adj16, wh_all, f1_all, f2t_all, out_w, oa1, oa2, br):
    heads, n, hid = wh_all.shape
    feat = out_w.shape[1]
    return pl.pallas_call(
        functools.partial(_k3_body, heads=heads),
        grid=(n // br,),
        in_specs=[
            pl.BlockSpec((br, n), lambda i: (i, 0)),
            pl.BlockSpec((heads, n, hid), lambda i: (0, 0, 0)),
            pl.BlockSpec((heads, br, 1), lambda i: (0, i, 0)),
            pl.BlockSpec((heads, 1, n), lambda i: (0, 0, 0)),
            pl.BlockSpec((heads * hid, feat), lambda i: (0, 0)),
            pl.BlockSpec((1, feat), lambda i: (0, 0)),
            pl.BlockSpec((1, feat), lambda i: (0, 0)),
        ],
        out_specs=[
            pl.BlockSpec((br, feat), lambda i: (i, 0)),
            pl.BlockSpec((br, 1), lambda i: (i, 0)),
            pl.BlockSpec((1, br), lambda i: (0, i)),
        ],
        out_shape=[
            jax.ShapeDtypeStruct((n, feat), jnp.float32),
            jax.ShapeDtypeStruct((n, 1), jnp.float32),
            jax.ShapeDtypeStruct((1, n), jnp.float32),
        ],
    )(adj16, wh_all, f1_all, f2t_all, out_w, oa1, oa2)


def _k4_body(adj16_ref, who_ref, f1o_ref, f2to_ref, out_ref):
    mask = adj16_ref[...] > 0
    att = _row_softmax_unnorm(f1o_ref[...] + f2to_ref[...], mask)
    hp = jnp.dot(att, who_ref[...], preferred_element_type=jnp.float32)
    y = jnp.where(hp > 0, hp, jnp.exp(hp) - 1.0)
    my = jnp.max(y, axis=1, keepdims=True)
    sh = y - my
    out_ref[...] = sh - jnp.log(jnp.sum(jnp.exp(sh), axis=1, keepdims=True))


def _k4(adj16, who, f1o, f2to, br):
    n, feat = who.shape
    return pl.pallas_call(
        _k4_body,
        grid=(n // br,),
        in_specs=[
            pl.BlockSpec((br, n), lambda i: (i, 0)),
            pl.BlockSpec((n, feat), lambda i: (0, 0)),
            pl.BlockSpec((br, 1), lambda i: (i, 0)),
            pl.BlockSpec((1, n), lambda i: (0, 0)),
        ],
        out_specs=pl.BlockSpec((br, feat), lambda i: (i, 0)),
        out_shape=jax.ShapeDtypeStruct((n, feat), jnp.float32),
    )(adj16, who, f1o, f2to)


def kernel(feature, adj, gcn_W1, gcn_b1, gcn_W2, gcn_b2, gat_W, gat_a,
           out_W, out_a):
    n, feat = feature.shape
    hid = gcn_W1.shape[1]
    heads = gat_W.shape[0]
    br = 128 if n % 128 == 0 else n
    brw = 512 if n % 512 == 0 else br

    b1 = gcn_b1.reshape(1, hid)
    b2 = gcn_b2.reshape(1, feat)
    a1 = gat_a[:, :hid, 0].reshape(heads, 1, hid)
    a2 = gat_a[:, hid:, 0].reshape(heads, 1, hid)
    oa1 = out_a[:feat, 0].reshape(1, feat)
    oa2 = out_a[feat:, 0].reshape(1, feat)

    x1, adj16 = _k1(feature, gcn_W1, b1, adj, brw)
    wh_all, f1_all, f2t_all = _k2(x1, gcn_W2, b2, gat_W, a1, a2, adj16, brw)
    who, f1o, f2to = _k3(adj16, wh_all, f1_all, f2t_all, out_W, oa1, oa2, br)
    x_out = _k4(adj16, who, f1o, f2to, brw)

    return (x_out, adj)


# speedup vs baseline: 1.1147x; 1.1147x over previous
import functools

import jax
import jax.numpy as jnp
from jax import lax
from jax.experimental import pallas as pl
from jax.experimental.pallas import tpu as pltpu

ALPHA = 0.1
NEG = -9e15


def _row_softmax_unnorm(e, eneg):
    e = jnp.maximum(e, ALPHA * e)
    e = e + eneg
    m = jnp.max(e, axis=1, keepdims=True)
    ex = jnp.exp(e - m)
    s = jnp.sum(ex, axis=1, keepdims=True)
    return ex * (1.0 / s)


def _k1_body(feat_ref, w1_ref, b1_ref, adj_ref, x1_ref, adj16_ref, p1_scr):
    @pl.when(pl.program_id(0) == 0)
    def _():
        p1_scr[...] = jnp.dot(feat_ref[...], w1_ref[...],
                              preferred_element_type=jnp.float32)

    a = adj_ref[...]
    adj16_ref[...] = a.astype(jnp.bfloat16)
    x = jnp.dot(a, p1_scr[...], preferred_element_type=jnp.float32)
    x1_ref[...] = jnp.maximum(x + b1_ref[...], 0.0)


def _k1(feature, w1, b1, adj, br):
    n, feat = feature.shape
    hid = w1.shape[1]
    return pl.pallas_call(
        _k1_body,
        grid=(n // br,),
        in_specs=[
            pl.BlockSpec((n, feat), lambda i: (0, 0)),
            pl.BlockSpec((feat, hid), lambda i: (0, 0)),
            pl.BlockSpec((1, hid), lambda i: (0, 0)),
            pl.BlockSpec((br, n), lambda i: (i, 0)),
        ],
        out_specs=[
            pl.BlockSpec((br, hid), lambda i: (i, 0)),
            pl.BlockSpec((br, n), lambda i: (i, 0)),
        ],
        out_shape=[
            jax.ShapeDtypeStruct((n, hid), jnp.float32),
            jax.ShapeDtypeStruct((n, n), jnp.bfloat16),
        ],
        scratch_shapes=[pltpu.VMEM((n, hid), jnp.float32)],
    )(feature, w1, b1, adj)


def _k2_body(x1_ref, w2_ref, b2_ref, gatw_ref, a1_ref, a2_ref, adj16_ref,
             wh_ref, f1_ref, f2t_ref, p2_scr, *, heads):
    @pl.when(pl.program_id(0) == 0)
    def _():
        p2_scr[...] = jnp.dot(x1_ref[...], w2_ref[...],
                              preferred_element_type=jnp.float32)

    a = adj16_ref[...].astype(jnp.float32)
    x2 = jnp.dot(a, p2_scr[...], preferred_element_type=jnp.float32)
    x2 = x2 + b2_ref[...]
    for h in range(heads):
        wh = jnp.dot(x2, gatw_ref[h], preferred_element_type=jnp.float32)
        wh_ref[h] = wh.astype(jnp.bfloat16)
        f1_ref[h] = lax.dot_general(wh, a1_ref[h], (((1,), (1,)), ((), ())),
                                    preferred_element_type=jnp.float32)
        f2t_ref[h] = lax.dot_general(a2_ref[h], wh, (((1,), (1,)), ((), ())),
                                     preferred_element_type=jnp.float32)


def _k2(x1, w2, b2, gat_w, a1, a2, adj16, br):
    n, hid = x1.shape
    heads, feat, ghid = gat_w.shape
    return pl.pallas_call(
        functools.partial(_k2_body, heads=heads),
        grid=(n // br,),
        in_specs=[
            pl.BlockSpec((n, hid), lambda i: (0, 0)),
            pl.BlockSpec((hid, feat), lambda i: (0, 0)),
            pl.BlockSpec((1, feat), lambda i: (0, 0)),
            pl.BlockSpec((heads, feat, ghid), lambda i: (0, 0, 0)),
            pl.BlockSpec((heads, 1, ghid), lambda i: (0, 0, 0)),
            pl.BlockSpec((heads, 1, ghid), lambda i: (0, 0, 0)),
            pl.BlockSpec((br, n), lambda i: (i, 0)),
        ],
        out_specs=[
            pl.BlockSpec((heads, br, ghid), lambda i: (0, i, 0)),
            pl.BlockSpec((heads, br, 1), lambda i: (0, i, 0)),
            pl.BlockSpec((heads, 1, br), lambda i: (0, 0, i)),
        ],
        out_shape=[
            jax.ShapeDtypeStruct((heads, n, ghid), jnp.bfloat16),
            jax.ShapeDtypeStruct((heads, n, 1), jnp.float32),
            jax.ShapeDtypeStruct((heads, 1, n), jnp.float32),
        ],
        scratch_shapes=[pltpu.VMEM((n, feat), jnp.float32)],
    )(x1, w2, b2, gat_w, a1, a2, adj16)


def _k3_body(adj16_ref, wh_ref, f1_ref, f2t_ref, outw_ref, oa1_ref, oa2_ref,
             who_ref, f1o_ref, f2to_ref, *, heads):
    eneg = (adj16_ref[...].astype(jnp.float32) - 1.0) * -NEG
    cats = []
    for h in range(heads):
        att = _row_softmax_unnorm(f1_ref[h] + f2t_ref[h], eneg)
        hp = jnp.dot(att, wh_ref[h], preferred_element_type=jnp.float32)
        cats.append(jnp.where(hp > 0, hp, jnp.exp(hp) - 1.0))
    xcat = jnp.concatenate(cats, axis=1)
    who = jnp.dot(xcat, outw_ref[...], preferred_element_type=jnp.float32)
    who_ref[...] = who
    f1o_ref[...] = lax.dot_general(who, oa1_ref[...], (((1,), (1,)), ((), ())),
                                   preferred_element_type=jnp.float32)
    f2to_ref[...] = lax.dot_general(oa2_ref[...], who, (((1,), (1,)), ((), ())),
                                    preferred_element_type=jnp.float32)


def _k3(adj16, wh_all, f1_all, f2t_all, out_w, oa1, oa2, br):
    heads, n, hid = wh_all.shape
    feat = out_w.shape[1]
    return pl.pallas_call(
        functools.partial(_k3_body, heads=heads),
        grid=(n // br,),
        in_specs=[
            pl.BlockSpec((br, n), lambda i: (i, 0)),
            pl.BlockSpec((heads, n, hid), lambda i: (0, 0, 0)),
            pl.BlockSpec((heads, br, 1), lambda i: (0, i, 0)),
            pl.BlockSpec((heads, 1, n), lambda i: (0, 0, 0)),
            pl.BlockSpec((heads * hid, feat), lambda i: (0, 0)),
            pl.BlockSpec((1, feat), lambda i: (0, 0)),
            pl.BlockSpec((1, feat), lambda i: (0, 0)),
        ],
        out_specs=[
            pl.BlockSpec((br, feat), lambda i: (i, 0)),
            pl.BlockSpec((br, 1), lambda i: (i, 0)),
            pl.BlockSpec((1, br), lambda i: (0, i)),
        ],
        out_shape=[
            jax.ShapeDtypeStruct((n, feat), jnp.float32),
            jax.ShapeDtypeStruct((n, 1), jnp.float32),
            jax.ShapeDtypeStruct((1, n), jnp.float32),
        ],
    )(adj16, wh_all, f1_all, f2t_all, out_w, oa1, oa2)


def _k4_body(adj16_ref, who_ref, f1o_ref, f2to_ref, out_ref):
    eneg = (adj16_ref[...].astype(jnp.float32) - 1.0) * -NEG
    att = _row_softmax_unnorm(f1o_ref[...] + f2to_ref[...], eneg)
    hp = jnp.dot(att, who_ref[...], preferred_element_type=jnp.float32)
    y = jnp.where(hp > 0, hp, jnp.exp(hp) - 1.0)
    my = jnp.max(y, axis=1, keepdims=True)
    sh = y - my
    out_ref[...] = sh - jnp.log(jnp.sum(jnp.exp(sh), axis=1, keepdims=True))


def _k4(adj16, who, f1o, f2to, br):
    n, feat = who.shape
    return pl.pallas_call(
        _k4_body,
        grid=(n // br,),
        in_specs=[
            pl.BlockSpec((br, n), lambda i: (i, 0)),
            pl.BlockSpec((n, feat), lambda i: (0, 0)),
            pl.BlockSpec((br, 1), lambda i: (i, 0)),
            pl.BlockSpec((1, n), lambda i: (0, 0)),
        ],
        out_specs=pl.BlockSpec((br, feat), lambda i: (i, 0)),
        out_shape=jax.ShapeDtypeStruct((n, feat), jnp.float32),
    )(adj16, who, f1o, f2to)


def kernel(feature, adj, gcn_W1, gcn_b1, gcn_W2, gcn_b2, gat_W, gat_a,
           out_W, out_a):
    n, feat = feature.shape
    hid = gcn_W1.shape[1]
    heads = gat_W.shape[0]
    br = 256 if n % 256 == 0 else n
    brw = 512 if n % 512 == 0 else br

    b1 = gcn_b1.reshape(1, hid)
    b2 = gcn_b2.reshape(1, feat)
    a1 = gat_a[:, :hid, 0].reshape(heads, 1, hid)
    a2 = gat_a[:, hid:, 0].reshape(heads, 1, hid)
    oa1 = out_a[:feat, 0].reshape(1, feat)
    oa2 = out_a[feat:, 0].reshape(1, feat)

    x1, adj16 = _k1(feature, gcn_W1, b1, adj, brw)
    wh_all, f1_all, f2t_all = _k2(x1, gcn_W2, b2, gat_W, a1, a2, adj16, brw)
    who, f1o, f2to = _k3(adj16, wh_all, f1_all, f2t_all, out_W, oa1, oa2, br)
    x_out = _k4(adj16, who, f1o, f2to, brw)

    return (x_out, adj)
